# baseline (speedup 1.0000x reference)
import functools

import jax
import jax.numpy as jnp
from jax import lax
from jax.experimental import pallas as pl
from jax.experimental.pallas import tpu as pltpu
from jax.experimental.pallas import tpu_sc as plsc

_N = 500000
_B = 16384
_D = 64
_H = 128

_NW = 32
_ROWS_PER_W = _B // _NW
_CHUNK = 128
_NCH = _ROWS_PER_W // _CHUNK


@functools.cache
def _make_gather_sc():
    mesh = plsc.VectorSubcoreMesh(core_axis_name="c", subcore_axis_name="s")

    @functools.partial(
        pl.kernel,
        mesh=mesh,
        out_type=(
            jax.ShapeDtypeStruct((_B // _CHUNK, _CHUNK, 2 * _D), jnp.float32),
            jax.ShapeDtypeStruct((_B // _CHUNK, _CHUNK, 2 * _D), jnp.float32),
        ),
        scratch_types=[
            pltpu.VMEM((_NCH, _CHUNK), jnp.int32),
            pltpu.VMEM((_NCH, _CHUNK), jnp.int32),
            pltpu.VMEM((_NCH, _CHUNK, 2 * _D), jnp.float32),
            pltpu.SemaphoreType.DMA,
        ],
    )
    def gather_sc(x_hbm, emb2_hbm, outf_hbm, outr_hbm, xf_v, xr_v, rows_v, sem):
        wid = lax.axis_index("s") * 2 + lax.axis_index("c")
        base = wid * _NCH
        pltpu.sync_copy(x_hbm.at[pl.ds(base, _NCH)], xf_v)
        for i in range(_NCH):
            for j in range(_CHUNK // 16):
                v = xf_v[i, pl.ds(j * 16, 16)]
                r = jnp.where(v >= _N, v - _N, v + _N)
                xf_v[i, pl.ds(j * 16, 16)] = lax.shift_right_logical(v, 1)
                xr_v[i, pl.ds(j * 16, 16)] = lax.shift_right_logical(r, 1)
        for idx_v, out_hbm in ((xf_v, outf_hbm), (xr_v, outr_hbm)):
            copies = []
            for i in range(_NCH):
                c = pltpu.make_async_copy(
                    emb2_hbm.at[idx_v.at[i]], rows_v.at[i], sem)
                c.start()
                copies.append(c)
            for c in copies:
                c.wait()
            pltpu.sync_copy(rows_v, out_hbm.at[pl.ds(base, _NCH)])

    return gather_sc


_BB = 2048


def _mlp_body(xf_ref, xr_ref, t_ref, xi_ref, w0s_ref, w0t_ref, b0_ref,
              w1_ref, b1_ref, w2_ref, b2_ref, wo_ref, out_ref):
    bf = jnp.bfloat16

    def dotbf(a, w):
        return jnp.dot(a.astype(bf), w.astype(bf),
                       preferred_element_type=jnp.float32)

    parity = lax.rem(xi_ref[...], 2)
    col = lax.broadcasted_iota(jnp.int32, (1, 2 * _D), 1)
    keep = (col < _D) == (parity == 0)

    t = jnp.maximum(t_ref[...], 0.0).astype(bf).astype(jnp.float32)
    w0t = w0t_ref[...].astype(bf).astype(jnp.float32)
    t_row = t * w0t + b0_ref[...]

    def branch(x):
        a = jnp.maximum(jnp.where(keep, x, 0.0), 0.0)
        h = dotbf(a, w0s_ref[...]) + t_row
        h = jnp.maximum(h, 0.0)
        h = dotbf(h, w1_ref[...]) + b1_ref[...]
        h = jnp.maximum(h, 0.0)
        h = dotbf(h, w2_ref[...]) + b2_ref[...]
        return jnp.maximum(h, 0.0).astype(bf).astype(jnp.float32)

    diff = branch(xf_ref[...]) - branch(xr_ref[...])
    wo = wo_ref[...].astype(bf).astype(jnp.float32)
    out_ref[...] = jnp.sum(diff * wo, axis=1, keepdims=True)


def _mlp_tc(xf, xr, t2, xi, w0s, w0t, b0r, w1, b1r, w2, b2r, wo_row):
    const = lambda i: (0, 0)
    return pl.pallas_call(
        _mlp_body,
        grid=(_B // _BB,),
        in_specs=[
            pl.BlockSpec((_BB, 2 * _D), lambda i: (i, 0)),
            pl.BlockSpec((_BB, 2 * _D), lambda i: (i, 0)),
            pl.BlockSpec((_BB, 1), lambda i: (i, 0)),
            pl.BlockSpec((_BB, 1), lambda i: (i, 0)),
            pl.BlockSpec((2 * _D, _H), const),
            pl.BlockSpec((1, _H), const),
            pl.BlockSpec((1, _H), const),
            pl.BlockSpec((_H, _H), const),
            pl.BlockSpec((1, _H), const),
            pl.BlockSpec((_H, _H), const),
            pl.BlockSpec((1, _H), const),
            pl.BlockSpec((1, _H), const),
        ],
        out_specs=pl.BlockSpec((_BB, 1), lambda i: (i, 0)),
        out_shape=jax.ShapeDtypeStruct((_B, 1), jnp.float32),
    )(xf, xr, t2, xi, w0s, w0t, b0r, w1, b1r, w2, b2r, wo_row)


def kernel(x, t, emb, W0, b0, W1, b1, W2, b2, Wo, bo):
    xi = x.astype(jnp.int32)
    x2 = xi.reshape(_B // _CHUNK, _CHUNK)
    emb2 = emb.reshape(_N, 2 * _D)
    outf, outr = _make_gather_sc()(x2, emb2)
    xf = outf.reshape(_B, 2 * _D)
    xr = outr.reshape(_B, 2 * _D)
    t2 = t[:, None]
    w0a = W0[:_D]
    w0s = jnp.concatenate([w0a, w0a], axis=0)
    w0t = W0[_D:_D + 1]
    out = _mlp_tc(xf, xr, t2, xi[:, None], w0s, w0t, b0[None, :],
                  W1, b1[None, :], W2, b2[None, :], Wo.T)
    return out


# speedup vs baseline: 1.1138x; 1.1138x over previous
import functools

import jax
import jax.numpy as jnp
from jax import lax
from jax.experimental import pallas as pl
from jax.experimental.pallas import tpu as pltpu
from jax.experimental.pallas import tpu_sc as plsc

_N = 500000
_B = 16384
_D = 64
_H = 128

_NW = 32
_ROWS_PER_W = _B // _NW
_CHUNK = 128
_NCH = _ROWS_PER_W // _CHUNK


@functools.cache
def _make_gather_sc():
    mesh = plsc.VectorSubcoreMesh(core_axis_name="c", subcore_axis_name="s")

    @functools.partial(
        pl.kernel,
        mesh=mesh,
        out_type=(
            jax.ShapeDtypeStruct((_B // _CHUNK, _CHUNK, 2 * _D), jnp.float32),
            jax.ShapeDtypeStruct((_B // _CHUNK, _CHUNK, 2 * _D), jnp.float32),
        ),
        scratch_types=[
            pltpu.VMEM((_NCH, _CHUNK), jnp.int32),
            pltpu.VMEM((_NCH, _CHUNK), jnp.int32),
            pltpu.VMEM((_NCH, _CHUNK, 2 * _D), jnp.float32),
            pltpu.SemaphoreType.DMA,
        ],
    )
    def gather_sc(x_hbm, emb_hbm, outf_hbm, outr_hbm, xf_v, xr_v, rows_v, sem):
        emb2_hbm = emb_hbm
        wid = lax.axis_index("s") * 2 + lax.axis_index("c")
        base = wid * _NCH
        pltpu.sync_copy(x_hbm.at[pl.ds(base, _NCH)], xf_v)
        for i in range(_NCH):
            for j in range(_CHUNK // 16):
                v = xf_v[i, pl.ds(j * 16, 16)]
                xr_v[i, pl.ds(j * 16, 16)] = jnp.where(v >= _N, v - _N, v + _N)
        for idx_v, out_hbm in ((xf_v, outf_hbm), (xr_v, outr_hbm)):
            copies = []
            for i in range(_NCH):
                c = pltpu.make_async_copy(
                    emb2_hbm.at[idx_v.at[i]], rows_v.at[i], sem)
                c.start()
                copies.append(c)
            for c in copies:
                c.wait()
            pltpu.sync_copy(rows_v, out_hbm.at[pl.ds(base, _NCH)])

    return gather_sc


_BB = 2048


def _mlp_body(xf_ref, xr_ref, t_ref, w0s_ref, w0t_ref, b0_ref,
              w1_ref, b1_ref, w2_ref, b2_ref, wo_ref, out_ref):
    bf = jnp.bfloat16

    def dotbf(a, w):
        return jnp.dot(a.astype(bf), w.astype(bf),
                       preferred_element_type=jnp.float32)

    col = lax.broadcasted_iota(jnp.int32, (1, 2 * _D), 1)
    keep = col < _D

    t = jnp.maximum(t_ref[...], 0.0).astype(bf).astype(jnp.float32)
    w0t = w0t_ref[...].astype(bf).astype(jnp.float32)
    t_row = t * w0t + b0_ref[...]

    def branch(x):
        a = jnp.maximum(jnp.where(keep, x, 0.0), 0.0)
        h = dotbf(a, w0s_ref[...]) + t_row
        h = jnp.maximum(h, 0.0)
        h = dotbf(h, w1_ref[...]) + b1_ref[...]
        h = jnp.maximum(h, 0.0)
        h = dotbf(h, w2_ref[...]) + b2_ref[...]
        return jnp.maximum(h, 0.0).astype(bf).astype(jnp.float32)

    diff = branch(xf_ref[...]) - branch(xr_ref[...])
    wo = wo_ref[...].astype(bf).astype(jnp.float32)
    out_ref[...] = jnp.sum(diff * wo, axis=1, keepdims=True)


def _mlp_tc(xf, xr, t2, w0s, w0t, b0r, w1, b1r, w2, b2r, wo_row):
    const = lambda i: (0, 0)
    return pl.pallas_call(
        _mlp_body,
        grid=(_B // _BB,),
        in_specs=[
            pl.BlockSpec((_BB, 2 * _D), lambda i: (i, 0)),
            pl.BlockSpec((_BB, 2 * _D), lambda i: (i, 0)),
            pl.BlockSpec((_BB, 1), lambda i: (i, 0)),
            pl.BlockSpec((2 * _D, _H), const),
            pl.BlockSpec((1, _H), const),
            pl.BlockSpec((1, _H), const),
            pl.BlockSpec((_H, _H), const),
            pl.BlockSpec((1, _H), const),
            pl.BlockSpec((_H, _H), const),
            pl.BlockSpec((1, _H), const),
            pl.BlockSpec((1, _H), const),
        ],
        out_specs=pl.BlockSpec((_BB, 1), lambda i: (i, 0)),
        out_shape=jax.ShapeDtypeStruct((_B, 1), jnp.float32),
    )(xf, xr, t2, w0s, w0t, b0r, w1, b1r, w2, b2r, wo_row)


def kernel(x, t, emb, W0, b0, W1, b1, W2, b2, Wo, bo):
    xi = x.astype(jnp.int32)
    x2 = xi.reshape(_B // _CHUNK, _CHUNK)
    embp = jnp.pad(emb, ((0, 0), (0, _D)))
    outf, outr = _make_gather_sc()(x2, embp)
    xf = outf.reshape(_B, 2 * _D)
    xr = outr.reshape(_B, 2 * _D)
    t2 = t[:, None]
    w0a = W0[:_D]
    w0s = jnp.concatenate([w0a, jnp.zeros_like(w0a)], axis=0)
    w0t = W0[_D:_D + 1]
    out = _mlp_tc(xf, xr, t2, w0s, w0t, b0[None, :],
                  W1, b1[None, :], W2, b2[None, :], Wo.T)
    return out
